# baseline (speedup 1.0000x reference)
import functools

import jax
import jax.numpy as jnp
from jax import lax
from jax.experimental import pallas as pl
from jax.experimental.pallas import tpu as pltpu
from jax.experimental.pallas import tpu_sc as plsc

N = 100000
D = 128
NSEG = 16
NC = 2
NS = 16
NW = NC * NS
CHUNK = 112
NCHUNK = 14
NBUF = 7
RPT = CHUNK * NCHUNK
STRIDE = RPT
LANES = 16
DBLK = D // LANES

TCB = 400
TC_K0 = 125
TC_NBLK = N // TCB - TC_K0

_mesh = plsc.VectorSubcoreMesh(
    core_axis_name="c", subcore_axis_name="s", num_cores=NC, num_subcores=NS
)


@functools.partial(
    pl.kernel,
    out_type=jax.ShapeDtypeStruct((NW, NSEG, D), jnp.float32),
    mesh=_mesh,
    scratch_types=[
        pltpu.VMEM((RPT,), jnp.int32),
        [pltpu.VMEM((CHUNK, D), jnp.float32) for _ in range(NBUF)],
        pltpu.VMEM((NSEG, D), jnp.float32),
        [pltpu.SemaphoreType.DMA for _ in range(NBUF)],
    ],
    compiler_params=pltpu.CompilerParams(needs_layout_passes=False),
)
def _sc_segmax(x_hbm, b_hbm, out_hbm, bv, bufs, accv, sems):
    wid = lax.axis_index("s") * NC + lax.axis_index("c")
    r0 = pl.multiple_of(wid * STRIDE, 8)

    for c in range(NBUF - 1):
        pltpu.async_copy(
            x_hbm.at[pl.ds(r0 + c * CHUNK, CHUNK)], bufs[c], sems[c]
        )
    pltpu.sync_copy(b_hbm.at[pl.ds(r0, RPT)], bv)

    seg_ids = lax.iota(jnp.int32, LANES)
    pos = jnp.zeros((LANES,), jnp.int32)
    step = 1024
    while step >= 1:
        cand = pos + step
        idx = jnp.minimum(cand, RPT) - 1
        vals = plsc.load_gather(bv, [idx])
        take = (cand <= RPT) & (vals < seg_ids)
        pos = jnp.where(take, cand, pos)
        step //= 2

    neg_inf = jnp.full((LANES,), -jnp.inf, jnp.float32)
    for s in range(NSEG):
        for j in range(DBLK):
            accv[s, pl.ds(j * LANES, LANES)] = neg_inf

    def process(c, mybuf, nxtbuf, mysem, nxtsem):
        pltpu.make_async_copy(x_hbm.at[pl.ds(0, CHUNK)], mybuf, mysem).wait()

        @pl.when(c + (NBUF - 1) < NCHUNK)
        def _():
            g = pl.multiple_of(r0 + (c + (NBUF - 1)) * CHUNK, 8)
            pltpu.async_copy(x_hbm.at[pl.ds(g, CHUNK)], nxtbuf, nxtsem)

        c0 = c * CHUNK
        sfirst = plsc.all_reduce_population_count(pos <= c0)[0] - 1
        slast = plsc.all_reduce_population_count(pos <= c0 + (CHUNK - 1))[0] - 1

        @pl.when(sfirst == slast)
        def _():
            def row_body(i, a):
                return tuple(
                    jnp.maximum(a[j], mybuf[i, pl.ds(j * LANES, LANES)])
                    for j in range(DBLK)
                )

            accs = lax.fori_loop(
                0, CHUNK, row_body, (neg_inf,) * DBLK, unroll=4
            )
            for j in range(DBLK):
                sl = pl.ds(j * LANES, LANES)
                accv[sfirst, sl] = jnp.maximum(accv[sfirst, sl], accs[j])

        @pl.when(sfirst != slast)
        def _():
            def grp_body(gi, carry):
                b16 = bv[pl.ds(c0 + gi * LANES, LANES)]
                for lane in range(LANES):
                    seg = b16[lane]
                    r = gi * LANES + lane
                    for j in range(DBLK):
                        sl = pl.ds(j * LANES, LANES)
                        accv[seg, sl] = jnp.maximum(accv[seg, sl], mybuf[r, sl])
                return carry

            lax.fori_loop(0, CHUNK // LANES, grp_body, 0)

    def ring_body(t, carry):
        for b in range(NBUF):
            c = NBUF * t + b
            nxt = (b + NBUF - 1) % NBUF
            process(c, bufs[b], bufs[nxt], sems[b], sems[nxt])
        return carry

    lax.fori_loop(0, NCHUNK // NBUF, ring_body, 0)

    pltpu.sync_copy(accv, out_hbm.at[wid])


def _tc_seg_body(b_ref, x_ref, o_ref):
    @pl.when(pl.program_id(0) == 0)
    def _():
        o_ref[...] = jnp.full((NSEG, D), -jnp.inf, jnp.float32)

    bcol = b_ref[0]
    sfirst = jnp.min(bcol)
    slast = jnp.max(bcol)
    xblk = x_ref[...]

    @pl.when(sfirst == slast)
    def _():
        mx = jnp.max(xblk, axis=0, keepdims=True)
        sl = pl.ds(sfirst, 1)
        o_ref[sl, :] = jnp.maximum(o_ref[sl, :], mx)

    @pl.when(sfirst != slast)
    def _():
        for s in range(NSEG):
            @pl.when(jnp.logical_and(s >= sfirst, s <= slast))
            def _(s=s):
                val = jnp.where(bcol == s, xblk, -jnp.inf)
                mx = jnp.max(val, axis=0, keepdims=True)
                sl = pl.ds(s, 1)
                o_ref[sl, :] = jnp.maximum(o_ref[sl, :], mx)


def _fold_body(p_ref, t_ref, o_ref):
    o_ref[...] = jnp.maximum(jnp.max(p_ref[...], axis=0), t_ref[...])


@jax.jit
def kernel(x, pos, batch):
    batch32 = batch.astype(jnp.int32)
    partial = _sc_segmax(x, batch32)
    tc_part = pl.pallas_call(
        _tc_seg_body,
        grid=(TC_NBLK,),
        in_specs=[
            pl.BlockSpec((1, TCB, 1), lambda i: (TC_K0 + i, 0, 0)),
            pl.BlockSpec((TCB, D), lambda i: (TC_K0 + i, 0)),
        ],
        out_specs=pl.BlockSpec((NSEG, D), lambda i: (0, 0)),
        out_shape=jax.ShapeDtypeStruct((NSEG, D), jnp.float32),
    )(batch32.reshape(N // TCB, TCB, 1), x)
    x_max = pl.pallas_call(
        _fold_body,
        out_shape=jax.ShapeDtypeStruct((NSEG, D), jnp.float32),
    )(partial, tc_part)
    pos_out = jnp.zeros((NSEG, 3), dtype=pos.dtype)
    batch_out = jnp.arange(NSEG, dtype=batch.dtype)
    return (x_max, pos_out, batch_out)


# speedup vs baseline: 1.5179x; 1.5179x over previous
import functools

import jax
import jax.numpy as jnp
from jax import lax
from jax.experimental import pallas as pl
from jax.experimental.pallas import tpu as pltpu
from jax.experimental.pallas import tpu_sc as plsc

N = 100000
D = 128
NSEG = 16
NC = 2
NS = 16
NW = NC * NS
CHUNK = 112
NCHUNK = 14
NBUF = 7
RPT = CHUNK * NCHUNK
STRIDE = RPT
LANES = 16
DBLK = D // LANES

TCB = 400
TC_K0 = 125
TC_NBLK = N // TCB - TC_K0

_mesh = plsc.VectorSubcoreMesh(
    core_axis_name="c", subcore_axis_name="s", num_cores=NC, num_subcores=NS
)


@functools.partial(
    pl.kernel,
    out_type=jax.ShapeDtypeStruct((NW, NSEG, D), jnp.float32),
    mesh=_mesh,
    scratch_types=[
        pltpu.VMEM((RPT,), jnp.int32),
        [pltpu.VMEM((CHUNK, D), jnp.float32) for _ in range(NBUF)],
        pltpu.VMEM((NSEG, D), jnp.float32),
        [pltpu.SemaphoreType.DMA for _ in range(NBUF)],
    ],
    compiler_params=pltpu.CompilerParams(needs_layout_passes=False),
)
def _sc_segmax(x_hbm, b_hbm, out_hbm, bv, bufs, accv, sems):
    wid = lax.axis_index("s") * NC + lax.axis_index("c")
    r0 = pl.multiple_of(wid * STRIDE, 8)

    for c in range(NBUF - 1):
        pltpu.async_copy(
            x_hbm.at[pl.ds(r0 + c * CHUNK, CHUNK)], bufs[c], sems[c]
        )
    pltpu.sync_copy(b_hbm.at[pl.ds(r0, RPT)], bv)

    seg_ids = lax.iota(jnp.int32, LANES)
    pos = jnp.zeros((LANES,), jnp.int32)
    step = 1024
    while step >= 1:
        cand = pos + step
        idx = jnp.minimum(cand, RPT) - 1
        vals = plsc.load_gather(bv, [idx])
        take = (cand <= RPT) & (vals < seg_ids)
        pos = jnp.where(take, cand, pos)
        step //= 2

    neg_inf = jnp.full((LANES,), -jnp.inf, jnp.float32)
    for s in range(NSEG):
        for j in range(DBLK):
            accv[s, pl.ds(j * LANES, LANES)] = neg_inf

    def process(c, mybuf, nxtbuf, mysem, nxtsem):
        pltpu.make_async_copy(x_hbm.at[pl.ds(0, CHUNK)], mybuf, mysem).wait()

        @pl.when(c + (NBUF - 1) < NCHUNK)
        def _():
            g = pl.multiple_of(r0 + (c + (NBUF - 1)) * CHUNK, 8)
            pltpu.async_copy(x_hbm.at[pl.ds(g, CHUNK)], nxtbuf, nxtsem)

        c0 = c * CHUNK
        sfirst = plsc.all_reduce_population_count(pos <= c0)[0] - 1
        slast = plsc.all_reduce_population_count(pos <= c0 + (CHUNK - 1))[0] - 1

        @pl.when(sfirst == slast)
        def _():
            def row_body(i, a):
                return tuple(
                    jnp.maximum(a[j], mybuf[i, pl.ds(j * LANES, LANES)])
                    for j in range(DBLK)
                )

            accs = lax.fori_loop(
                0, CHUNK, row_body, (neg_inf,) * DBLK, unroll=4
            )
            for j in range(DBLK):
                sl = pl.ds(j * LANES, LANES)
                accv[sfirst, sl] = jnp.maximum(accv[sfirst, sl], accs[j])

        @pl.when(sfirst != slast)
        def _():
            def grp_body(gi, carry):
                b16 = bv[pl.ds(c0 + gi * LANES, LANES)]
                for lane in range(LANES):
                    seg = b16[lane]
                    r = gi * LANES + lane
                    for j in range(DBLK):
                        sl = pl.ds(j * LANES, LANES)
                        accv[seg, sl] = jnp.maximum(accv[seg, sl], mybuf[r, sl])
                return carry

            lax.fori_loop(0, CHUNK // LANES, grp_body, 0)

    def ring_body(t, carry):
        for b in range(NBUF):
            c = NBUF * t + b
            nxt = (b + NBUF - 1) % NBUF
            process(c, bufs[b], bufs[nxt], sems[b], sems[nxt])
        return carry

    lax.fori_loop(0, NCHUNK // NBUF, ring_body, 0)

    pltpu.sync_copy(accv, out_hbm.at[wid])


def _tc_seg_body(b_ref, x_ref, o_ref):
    @pl.when(pl.program_id(0) == 0)
    def _():
        o_ref[...] = jnp.full((NSEG, D), -jnp.inf, jnp.float32)

    bcol = b_ref[0]
    sfirst = jnp.min(bcol)
    slast = jnp.max(bcol)
    xblk = x_ref[...]

    @pl.when(sfirst == slast)
    def _():
        mx = jnp.max(xblk, axis=0, keepdims=True)
        sl = pl.ds(sfirst, 1)
        o_ref[sl, :] = jnp.maximum(o_ref[sl, :], mx)

    @pl.when(sfirst != slast)
    def _():
        rows2d = lax.broadcasted_iota(jnp.int32, (TCB, D), 0)
        for s in range(NSEG):
            lo = jnp.sum((bcol < s).astype(jnp.int32))
            hi = jnp.sum((bcol <= s).astype(jnp.int32))

            @pl.when(hi > lo)
            def _(s=s, lo=lo, hi=hi):
                val = jnp.where(
                    (rows2d >= lo) & (rows2d < hi), xblk, -jnp.inf
                )
                mx = jnp.max(val, axis=0, keepdims=True)
                sl = pl.ds(s, 1)
                o_ref[sl, :] = jnp.maximum(o_ref[sl, :], mx)


def _fold_body(p_ref, t_ref, o_ref):
    o_ref[...] = jnp.maximum(jnp.max(p_ref[...], axis=0), t_ref[...])


@jax.jit
def kernel(x, pos, batch):
    batch32 = batch.astype(jnp.int32)
    partial = _sc_segmax(x, batch32)
    tc_part = pl.pallas_call(
        _tc_seg_body,
        grid=(TC_NBLK,),
        in_specs=[
            pl.BlockSpec((1, 1, TCB), lambda i: (TC_K0 + i, 0, 0)),
            pl.BlockSpec((TCB, D), lambda i: (TC_K0 + i, 0)),
        ],
        out_specs=pl.BlockSpec((NSEG, D), lambda i: (0, 0)),
        out_shape=jax.ShapeDtypeStruct((NSEG, D), jnp.float32),
    )(batch32.reshape(N // TCB, 1, TCB), x)
    x_max = pl.pallas_call(
        _fold_body,
        out_shape=jax.ShapeDtypeStruct((NSEG, D), jnp.float32),
    )(partial, tc_part)
    pos_out = jnp.zeros((NSEG, 3), dtype=pos.dtype)
    batch_out = jnp.arange(NSEG, dtype=batch.dtype)
    return (x_max, pos_out, batch_out)


# speedup vs baseline: 2.7555x; 1.8153x over previous
import functools

import jax
import jax.numpy as jnp
from jax import lax
from jax.experimental import pallas as pl
from jax.experimental.pallas import tpu as pltpu
from jax.experimental.pallas import tpu_sc as plsc

N = 100000
D = 128
NSEG = 16
NC = 2
NS = 16
NW = NC * NS
CHUNK = 112
NCHUNK = 14
NBUF = 7
RPT = CHUNK * NCHUNK
STRIDE = RPT
LANES = 16
DBLK = D // LANES

TCB = 2000
TC_K0 = 25
TC_NBLK = N // TCB - TC_K0

_mesh = plsc.VectorSubcoreMesh(
    core_axis_name="c", subcore_axis_name="s", num_cores=NC, num_subcores=NS
)


@functools.partial(
    pl.kernel,
    out_type=jax.ShapeDtypeStruct((NW, NSEG, D), jnp.float32),
    mesh=_mesh,
    scratch_types=[
        pltpu.VMEM((RPT,), jnp.int32),
        [pltpu.VMEM((CHUNK, D), jnp.float32) for _ in range(NBUF)],
        pltpu.VMEM((NSEG, D), jnp.float32),
        [pltpu.SemaphoreType.DMA for _ in range(NBUF)],
    ],
    compiler_params=pltpu.CompilerParams(needs_layout_passes=False),
)
def _sc_segmax(x_hbm, b_hbm, out_hbm, bv, bufs, accv, sems):
    wid = lax.axis_index("s") * NC + lax.axis_index("c")
    r0 = pl.multiple_of(wid * STRIDE, 8)

    for c in range(NBUF - 1):
        pltpu.async_copy(
            x_hbm.at[pl.ds(r0 + c * CHUNK, CHUNK)], bufs[c], sems[c]
        )
    pltpu.sync_copy(b_hbm.at[pl.ds(r0, RPT)], bv)

    seg_ids = lax.iota(jnp.int32, LANES)
    pos = jnp.zeros((LANES,), jnp.int32)
    step = 1024
    while step >= 1:
        cand = pos + step
        idx = jnp.minimum(cand, RPT) - 1
        vals = plsc.load_gather(bv, [idx])
        take = (cand <= RPT) & (vals < seg_ids)
        pos = jnp.where(take, cand, pos)
        step //= 2

    neg_inf = jnp.full((LANES,), -jnp.inf, jnp.float32)
    for s in range(NSEG):
        for j in range(DBLK):
            accv[s, pl.ds(j * LANES, LANES)] = neg_inf

    def process(c, mybuf, nxtbuf, mysem, nxtsem):
        pltpu.make_async_copy(x_hbm.at[pl.ds(0, CHUNK)], mybuf, mysem).wait()

        @pl.when(c + (NBUF - 1) < NCHUNK)
        def _():
            g = pl.multiple_of(r0 + (c + (NBUF - 1)) * CHUNK, 8)
            pltpu.async_copy(x_hbm.at[pl.ds(g, CHUNK)], nxtbuf, nxtsem)

        c0 = c * CHUNK
        sfirst = plsc.all_reduce_population_count(pos <= c0)[0] - 1
        slast = plsc.all_reduce_population_count(pos <= c0 + (CHUNK - 1))[0] - 1

        @pl.when(sfirst == slast)
        def _():
            def row_body(i, a):
                return tuple(
                    jnp.maximum(a[j], mybuf[i, pl.ds(j * LANES, LANES)])
                    for j in range(DBLK)
                )

            accs = lax.fori_loop(
                0, CHUNK, row_body, (neg_inf,) * DBLK, unroll=4
            )
            for j in range(DBLK):
                sl = pl.ds(j * LANES, LANES)
                accv[sfirst, sl] = jnp.maximum(accv[sfirst, sl], accs[j])

        @pl.when(sfirst != slast)
        def _():
            def grp_body(gi, carry):
                b16 = bv[pl.ds(c0 + gi * LANES, LANES)]
                for lane in range(LANES):
                    seg = b16[lane]
                    r = gi * LANES + lane
                    for j in range(DBLK):
                        sl = pl.ds(j * LANES, LANES)
                        accv[seg, sl] = jnp.maximum(accv[seg, sl], mybuf[r, sl])
                return carry

            lax.fori_loop(0, CHUNK // LANES, grp_body, 0)

    def ring_body(t, carry):
        for b in range(NBUF):
            c = NBUF * t + b
            nxt = (b + NBUF - 1) % NBUF
            process(c, bufs[b], bufs[nxt], sems[b], sems[nxt])
        return carry

    lax.fori_loop(0, NCHUNK // NBUF, ring_body, 0)

    pltpu.sync_copy(accv, out_hbm.at[wid])


def _tc_seg_body(b_ref, x_ref, o_ref):
    @pl.when(pl.program_id(0) == 0)
    def _():
        o_ref[...] = jnp.full((NSEG, D), -jnp.inf, jnp.float32)

    bcol = b_ref[0]
    sfirst = jnp.min(bcol)
    slast = jnp.max(bcol)
    xblk = x_ref[...]

    @pl.when(sfirst == slast)
    def _():
        mx = jnp.max(xblk, axis=0, keepdims=True)
        sl = pl.ds(sfirst, 1)
        o_ref[sl, :] = jnp.maximum(o_ref[sl, :], mx)

    @pl.when(sfirst != slast)
    def _():
        rows2d = lax.broadcasted_iota(jnp.int32, (TCB, D), 0)
        for s in range(NSEG):
            lo = jnp.sum((bcol < s).astype(jnp.int32))
            hi = jnp.sum((bcol <= s).astype(jnp.int32))

            @pl.when(hi > lo)
            def _(s=s, lo=lo, hi=hi):
                val = jnp.where(
                    (rows2d >= lo) & (rows2d < hi), xblk, -jnp.inf
                )
                mx = jnp.max(val, axis=0, keepdims=True)
                sl = pl.ds(s, 1)
                o_ref[sl, :] = jnp.maximum(o_ref[sl, :], mx)


def _fold_body(p_ref, t_ref, o_ref):
    o_ref[...] = jnp.maximum(jnp.max(p_ref[...], axis=0), t_ref[...])


@jax.jit
def kernel(x, pos, batch):
    batch32 = batch.astype(jnp.int32)
    partial = _sc_segmax(x, batch32)
    tc_part = pl.pallas_call(
        _tc_seg_body,
        grid=(TC_NBLK,),
        in_specs=[
            pl.BlockSpec((1, 1, TCB), lambda i: (TC_K0 + i, 0, 0)),
            pl.BlockSpec((TCB, D), lambda i: (TC_K0 + i, 0)),
        ],
        out_specs=pl.BlockSpec((NSEG, D), lambda i: (0, 0)),
        out_shape=jax.ShapeDtypeStruct((NSEG, D), jnp.float32),
    )(batch32.reshape(N // TCB, 1, TCB), x)
    x_max = pl.pallas_call(
        _fold_body,
        out_shape=jax.ShapeDtypeStruct((NSEG, D), jnp.float32),
    )(partial, tc_part)
    pos_out = jnp.zeros((NSEG, 3), dtype=pos.dtype)
    batch_out = jnp.arange(NSEG, dtype=batch.dtype)
    return (x_max, pos_out, batch_out)


# speedup vs baseline: 2.9968x; 1.0876x over previous
import functools

import jax
import jax.numpy as jnp
from jax import lax
from jax.experimental import pallas as pl
from jax.experimental.pallas import tpu as pltpu
from jax.experimental.pallas import tpu_sc as plsc

N = 100000
D = 128
NSEG = 16
NC = 2
NS = 16
NW = NC * NS
CHUNK = 112
NCHUNK = 28
NBUF = 7
RPT = CHUNK * NCHUNK
STRIDE = 3128
LANES = 16
DBLK = D // LANES

_mesh = plsc.VectorSubcoreMesh(
    core_axis_name="c", subcore_axis_name="s", num_cores=NC, num_subcores=NS
)


@functools.partial(
    pl.kernel,
    out_type=jax.ShapeDtypeStruct((NW, NSEG, D), jnp.float32),
    mesh=_mesh,
    scratch_types=[
        pltpu.VMEM((RPT,), jnp.int32),
        [pltpu.VMEM((CHUNK, D), jnp.float32) for _ in range(NBUF)],
        pltpu.VMEM((NSEG, D), jnp.float32),
        [pltpu.SemaphoreType.DMA for _ in range(NBUF)],
    ],
    compiler_params=pltpu.CompilerParams(needs_layout_passes=False),
)
def _sc_segmax(x_hbm, b_hbm, out_hbm, bv, bufs, accv, sems):
    wid = lax.axis_index("s") * NC + lax.axis_index("c")
    r0 = pl.multiple_of(jnp.minimum(wid * STRIDE, N - RPT), 8)

    for c in range(NBUF - 1):
        pltpu.async_copy(
            x_hbm.at[pl.ds(r0 + c * CHUNK, CHUNK)], bufs[c], sems[c]
        )
    pltpu.sync_copy(b_hbm.at[pl.ds(r0, RPT)], bv)

    seg_ids = lax.iota(jnp.int32, LANES)
    pos = jnp.zeros((LANES,), jnp.int32)
    step = 2048
    while step >= 1:
        cand = pos + step
        idx = jnp.minimum(cand, RPT) - 1
        vals = plsc.load_gather(bv, [idx])
        take = (cand <= RPT) & (vals < seg_ids)
        pos = jnp.where(take, cand, pos)
        step //= 2

    neg_inf = jnp.full((LANES,), -jnp.inf, jnp.float32)
    for s in range(NSEG):
        for j in range(DBLK):
            accv[s, pl.ds(j * LANES, LANES)] = neg_inf

    def process(c, mybuf, nxtbuf, mysem, nxtsem):
        pltpu.make_async_copy(x_hbm.at[pl.ds(0, CHUNK)], mybuf, mysem).wait()

        @pl.when(c + (NBUF - 1) < NCHUNK)
        def _():
            g = pl.multiple_of(r0 + (c + (NBUF - 1)) * CHUNK, 8)
            pltpu.async_copy(x_hbm.at[pl.ds(g, CHUNK)], nxtbuf, nxtsem)

        c0 = c * CHUNK
        sfirst = plsc.all_reduce_population_count(pos <= c0)[0] - 1
        slast = plsc.all_reduce_population_count(pos <= c0 + (CHUNK - 1))[0] - 1

        @pl.when(sfirst == slast)
        def _():
            def row_body(i, a):
                return tuple(
                    jnp.maximum(a[j], mybuf[i, pl.ds(j * LANES, LANES)])
                    for j in range(DBLK)
                )

            accs = lax.fori_loop(
                0, CHUNK, row_body, (neg_inf,) * DBLK, unroll=4
            )
            for j in range(DBLK):
                sl = pl.ds(j * LANES, LANES)
                accv[sfirst, sl] = jnp.maximum(accv[sfirst, sl], accs[j])

        @pl.when(sfirst != slast)
        def _():
            def grp_body(gi, carry):
                b16 = bv[pl.ds(c0 + gi * LANES, LANES)]
                for lane in range(LANES):
                    seg = b16[lane]
                    r = gi * LANES + lane
                    for j in range(DBLK):
                        sl = pl.ds(j * LANES, LANES)
                        accv[seg, sl] = jnp.maximum(accv[seg, sl], mybuf[r, sl])
                return carry

            lax.fori_loop(0, CHUNK // LANES, grp_body, 0)

    def ring_body(t, carry):
        for b in range(NBUF):
            c = NBUF * t + b
            nxt = (b + NBUF - 1) % NBUF
            process(c, bufs[b], bufs[nxt], sems[b], sems[nxt])
        return carry

    lax.fori_loop(0, NCHUNK // NBUF, ring_body, 0)

    pltpu.sync_copy(accv, out_hbm.at[wid])


def _fold_body(p_ref, o_ref):
    o_ref[...] = jnp.max(p_ref[...], axis=0)


@jax.jit
def kernel(x, pos, batch):
    batch32 = batch.astype(jnp.int32)
    partial = _sc_segmax(x, batch32)
    x_max = pl.pallas_call(
        _fold_body,
        out_shape=jax.ShapeDtypeStruct((NSEG, D), jnp.float32),
    )(partial)
    pos_out = jnp.zeros((NSEG, 3), dtype=pos.dtype)
    batch_out = jnp.arange(NSEG, dtype=batch.dtype)
    return (x_max, pos_out, batch_out)
